# baseline (speedup 1.0000x reference)
import functools

import jax
import jax.numpy as jnp
from jax import lax
from jax.experimental import pallas as pl
from jax.experimental.pallas import tpu as pltpu
from jax.experimental.pallas import tpu_sc as plsc

_N = 10000
_E = 320000
_D = 128

_NC = 2
_NS = 16
_NW = _NC * _NS
_STR = 624
_TAIL0 = _STR * _NS
_TAIL = _N - _TAIL0

_sc_mesh = plsc.VectorSubcoreMesh(core_axis_name="c", subcore_axis_name="s")

_CH = 128
_CPT = 80
_EPAD = _NW * _CPT * _CH
_NROW_CHUNKS = _EPAD // _CH
_ACC_ROWS = _N + 16
_NBUF = 2
_CPH = 40

_DEG_W = 16
_DEG_NBUF = 8


def _deg_body(dst_hbm, ones_hbm, zeros_hbm, deg_out, idx_v, ones_v, acc_sh,
              *sems):
    cid = lax.axis_index("c")
    sid = lax.axis_index("s")
    wid = sid * _NC + cid
    r0 = sid * _STR

    pltpu.sync_copy(dst_hbm.at[pl.ds(wid * _CPT, _CPT)], idx_v)
    pltpu.sync_copy(ones_hbm, ones_v)
    pltpu.sync_copy(zeros_hbm.at[pl.ds(r0, _STR)], acc_sh.at[pl.ds(r0, _STR)])

    @pl.when(sid == 0)
    def _():
        pltpu.sync_copy(zeros_hbm.at[pl.ds(_TAIL0, _TAIL)],
                        acc_sh.at[pl.ds(_TAIL0, _TAIL)])

    plsc.subcore_barrier()

    def group(g, carry):
        for b in range(_DEG_NBUF):
            c = g * _DEG_NBUF + b

            @pl.when(g > 0)
            def _():
                pltpu.make_async_copy(
                    ones_v, acc_sh.at[idx_v.at[c - _DEG_NBUF]], sems[b]).wait()

            pltpu.async_copy(ones_v, acc_sh.at[idx_v.at[c]], sems[b], add=True)
        return carry

    lax.fori_loop(0, _CPT // _DEG_NBUF, group, 0)
    for b in range(_DEG_NBUF):
        c = _CPT - _DEG_NBUF + b
        pltpu.make_async_copy(ones_v, acc_sh.at[idx_v.at[c]], sems[b]).wait()

    plsc.subcore_barrier()
    pltpu.sync_copy(acc_sh.at[pl.ds(r0, _STR)],
                    deg_out.at[cid, pl.ds(r0, _STR)])

    @pl.when(sid == 0)
    def _():
        pltpu.sync_copy(acc_sh.at[pl.ds(_TAIL0, _TAIL)],
                        deg_out.at[cid, pl.ds(_TAIL0, _TAIL)])


_deg_kernel = pl.kernel(
    _deg_body,
    out_type=jax.ShapeDtypeStruct((_NC, _N, _DEG_W), jnp.float32),
    mesh=_sc_mesh,
    scratch_types=[
        pltpu.VMEM((_CPT, _CH), jnp.int32),
        pltpu.VMEM((_CH, _DEG_W), jnp.float32),
        pltpu.VMEM_SHARED((_ACC_ROWS, _DEG_W), jnp.float32),
    ] + [pltpu.SemaphoreType.DMA] * _DEG_NBUF,
)


def _spmm_body(hs_hbm, src_hbm, dst_hbm, zeros_hbm, out_hbm,
               src_v, dst_v, rows_v, acc_sh, *sems):
    cid = lax.axis_index("c")
    sid = lax.axis_index("s")
    wid = sid * _NC + cid
    r0 = sid * _STR
    sg = sems[:_NBUF]
    ss = sems[_NBUF:]

    pltpu.sync_copy(zeros_hbm.at[pl.ds(r0, _STR)], acc_sh.at[pl.ds(r0, _STR)])

    @pl.when(sid == 0)
    def _():
        pltpu.sync_copy(zeros_hbm.at[pl.ds(_TAIL0, _TAIL)],
                        acc_sh.at[pl.ds(_TAIL0, _TAIL)])

    plsc.subcore_barrier()

    for phase in range(_CPT // _CPH):
        base = wid * _CPT + phase * _CPH
        pltpu.sync_copy(src_hbm.at[pl.ds(base, _CPH)], src_v)
        pltpu.sync_copy(dst_hbm.at[pl.ds(base, _CPH)], dst_v)

        def group(g, carry):
            for b in range(_NBUF):
                c = g * _NBUF + b

                @pl.when(g > 0)
                def _():
                    pltpu.make_async_copy(
                        rows_v.at[b], acc_sh.at[dst_v.at[c - _NBUF]],
                        ss[b]).wait()

                pltpu.async_copy(hs_hbm.at[src_v.at[c]], rows_v.at[b], sg[b])
            for b in range(_NBUF):
                c = g * _NBUF + b
                pltpu.make_async_copy(hs_hbm.at[src_v.at[c]], rows_v.at[b],
                                      sg[b]).wait()
                pltpu.async_copy(rows_v.at[b], acc_sh.at[dst_v.at[c]], ss[b],
                                 add=True)
            return carry

        lax.fori_loop(0, _CPH // _NBUF, group, 0)
        for b in range(_NBUF):
            c = _CPH - _NBUF + b
            pltpu.make_async_copy(rows_v.at[b], acc_sh.at[dst_v.at[c]],
                                  ss[b]).wait()

    plsc.subcore_barrier()
    pltpu.sync_copy(acc_sh.at[pl.ds(r0, _STR)],
                    out_hbm.at[cid, pl.ds(r0, _STR)])

    @pl.when(sid == 0)
    def _():
        pltpu.sync_copy(acc_sh.at[pl.ds(_TAIL0, _TAIL)],
                        out_hbm.at[cid, pl.ds(_TAIL0, _TAIL)])


_spmm_kernel = pl.kernel(
    _spmm_body,
    out_type=jax.ShapeDtypeStruct((_NC, _N, _D), jnp.float32),
    mesh=_sc_mesh,
    scratch_types=[
        pltpu.VMEM((_CPH, _CH), jnp.int32),
        pltpu.VMEM((_CPH, _CH), jnp.int32),
        pltpu.VMEM((_NBUF, _CH, _D), jnp.float32),
        pltpu.VMEM_SHARED((_ACC_ROWS, _D), jnp.float32),
    ] + [pltpu.SemaphoreType.DMA] * (2 * _NBUF),
)

_RB = 2000
_NB = _N // _RB


def _pre_body(deg_ref, x_ref, dinv_ref, xs_ref):
    deg = deg_ref[0, :, :1] + deg_ref[1, :, :1] + 1.0
    dinv = lax.rsqrt(deg)
    dinv_ref[...] = dinv
    xs_ref[...] = x_ref[...] * dinv


_pre_call = pl.pallas_call(
    _pre_body,
    out_shape=[
        jax.ShapeDtypeStruct((_N, 1), jnp.float32),
        jax.ShapeDtypeStruct((_N, _D), jnp.float32),
    ],
)


def _mm1_body(parts_ref, self_ref, dinv_ref, w_ref, b_ref, t_ref, sums_ref):
    p = (parts_ref[0] + parts_ref[1] + self_ref[...]) * dinv_ref[...]
    t = jnp.dot(p, w_ref[...], preferred_element_type=jnp.float32) + b_ref[...]
    t = jnp.maximum(t, 0.0)
    t_ref[...] = t
    sums_ref[...] = jnp.stack([jnp.sum(t, axis=0), jnp.sum(t * t, axis=0)])[None]


def _mk_mm1(wout):
    return pl.pallas_call(
        _mm1_body,
        grid=(_NB,),
        in_specs=[
            pl.BlockSpec((_NC, _RB, _D), lambda i: (0, i, 0)),
            pl.BlockSpec((_RB, _D), lambda i: (i, 0)),
            pl.BlockSpec((_RB, 1), lambda i: (i, 0)),
            pl.BlockSpec((_D, wout), lambda i: (0, 0)),
            pl.BlockSpec((1, wout), lambda i: (0, 0)),
        ],
        out_specs=[
            pl.BlockSpec((_RB, wout), lambda i: (i, 0)),
            pl.BlockSpec((1, 2, wout), lambda i: (i, 0, 0)),
        ],
        out_shape=[
            jax.ShapeDtypeStruct((_N, wout), jnp.float32),
            jax.ShapeDtypeStruct((_NB, 2, wout), jnp.float32),
        ],
    )


_mm_128 = _mk_mm1(_D)
_mm_256 = _mk_mm1(2 * _D)


def _mm2_body(pa_ref, pb_ref, sa_ref, sb_ref, dinv_ref, w_ref, b_ref,
              t_ref, sums_ref):
    dinv = dinv_ref[...]
    pa = (pa_ref[0] + pa_ref[1] + sa_ref[...]) * dinv
    pb = (pb_ref[0] + pb_ref[1] + sb_ref[...]) * dinv
    p = jnp.concatenate([pa, pb], axis=1)
    t = jnp.dot(p, w_ref[...], preferred_element_type=jnp.float32) + b_ref[...]
    t = jnp.maximum(t, 0.0)
    t_ref[...] = t
    sums_ref[...] = jnp.stack([jnp.sum(t, axis=0), jnp.sum(t * t, axis=0)])[None]


_mm2_256 = pl.pallas_call(
    _mm2_body,
    grid=(_NB,),
    in_specs=[
        pl.BlockSpec((_NC, _RB, _D), lambda i: (0, i, 0)),
        pl.BlockSpec((_NC, _RB, _D), lambda i: (0, i, 0)),
        pl.BlockSpec((_RB, _D), lambda i: (i, 0)),
        pl.BlockSpec((_RB, _D), lambda i: (i, 0)),
        pl.BlockSpec((_RB, 1), lambda i: (i, 0)),
        pl.BlockSpec((2 * _D, 2 * _D), lambda i: (0, 0)),
        pl.BlockSpec((1, 2 * _D), lambda i: (0, 0)),
    ],
    out_specs=[
        pl.BlockSpec((_RB, 2 * _D), lambda i: (i, 0)),
        pl.BlockSpec((1, 2, 2 * _D), lambda i: (i, 0, 0)),
    ],
    out_shape=[
        jax.ShapeDtypeStruct((_N, 2 * _D), jnp.float32),
        jax.ShapeDtypeStruct((_NB, 2, 2 * _D), jnp.float32),
    ],
)


def _bn_core(t_ref, sums_ref, g_ref, be_ref):
    s = jnp.sum(sums_ref[...], axis=0)
    m = s[0] * (1.0 / _N)
    v = s[1] * (1.0 / _N) - m * m
    scale = g_ref[...] * lax.rsqrt(v + 1e-5)[None, :]
    return (t_ref[...] - m[None, :]) * scale + be_ref[...]


def _bn_scale_body(t_ref, sums_ref, g_ref, be_ref, dinv_ref, o_ref):
    o_ref[...] = _bn_core(t_ref, sums_ref, g_ref, be_ref) * dinv_ref[...]


def _bn_split_body(t_ref, sums_ref, g_ref, be_ref, dinv_ref, oa_ref, ob_ref):
    h = _bn_core(t_ref, sums_ref, g_ref, be_ref) * dinv_ref[...]
    oa_ref[...] = h[:, :_D]
    ob_ref[...] = h[:, _D:]


def _bn_final_body(t_ref, sums_ref, g_ref, be_ref, o_ref):
    o_ref[...] = _bn_core(t_ref, sums_ref, g_ref, be_ref)


def _bn_in_specs(wout, with_dinv):
    specs = [
        pl.BlockSpec((_RB, wout), lambda i: (i, 0)),
        pl.BlockSpec((_NB, 2, wout), lambda i: (0, 0, 0)),
        pl.BlockSpec((1, wout), lambda i: (0, 0)),
        pl.BlockSpec((1, wout), lambda i: (0, 0)),
    ]
    if with_dinv:
        specs.append(pl.BlockSpec((_RB, 1), lambda i: (i, 0)))
    return specs


_bn_scale_128 = pl.pallas_call(
    _bn_scale_body,
    grid=(_NB,),
    in_specs=_bn_in_specs(_D, True),
    out_specs=pl.BlockSpec((_RB, _D), lambda i: (i, 0)),
    out_shape=jax.ShapeDtypeStruct((_N, _D), jnp.float32),
)

_bn_split_256 = pl.pallas_call(
    _bn_split_body,
    grid=(_NB,),
    in_specs=_bn_in_specs(2 * _D, True),
    out_specs=[
        pl.BlockSpec((_RB, _D), lambda i: (i, 0)),
        pl.BlockSpec((_RB, _D), lambda i: (i, 0)),
    ],
    out_shape=[
        jax.ShapeDtypeStruct((_N, _D), jnp.float32),
        jax.ShapeDtypeStruct((_N, _D), jnp.float32),
    ],
)

_bn_final_256 = pl.pallas_call(
    _bn_final_body,
    grid=(_NB,),
    in_specs=_bn_in_specs(2 * _D, False),
    out_specs=pl.BlockSpec((_RB, 2 * _D), lambda i: (i, 0)),
    out_shape=jax.ShapeDtypeStruct((_N, 2 * _D), jnp.float32),
)


def kernel(x, edge_index, W1, b1, g1, be1, W2, b2, g2, be2, W3, b3, g3, be3):
    pad = _EPAD - _E
    src = jnp.concatenate([edge_index[0],
                           jnp.zeros((pad,), jnp.int32)]).reshape(
                               _NROW_CHUNKS, _CH)
    dst = jnp.concatenate([edge_index[1],
                           jnp.full((pad,), _N, jnp.int32)]).reshape(
                               _NROW_CHUNKS, _CH)

    zeros = jnp.zeros((_N, _D), jnp.float32)
    ones16 = jnp.ones((_CH, _DEG_W), jnp.float32)
    deg_parts = _deg_kernel(dst, ones16, zeros[:, :_DEG_W])
    dinv, xs = _pre_call(deg_parts, x)

    s0 = _spmm_kernel(xs, src, dst, zeros)
    t1, sums1 = _mm_128(s0, xs, dinv, W1, b1.reshape(1, -1))
    hs1 = _bn_scale_128(t1, sums1, g1.reshape(1, -1), be1.reshape(1, -1), dinv)

    s1 = _spmm_kernel(hs1, src, dst, zeros)
    t2, sums2 = _mm_256(s1, hs1, dinv, W2, b2.reshape(1, -1))
    hs2a, hs2b = _bn_split_256(t2, sums2, g2.reshape(1, -1),
                               be2.reshape(1, -1), dinv)

    s2a = _spmm_kernel(hs2a, src, dst, zeros)
    s2b = _spmm_kernel(hs2b, src, dst, zeros)
    t3, sums3 = _mm2_256(s2a, s2b, hs2a, hs2b, dinv, W3, b3.reshape(1, -1))
    out = _bn_final_256(t3, sums3, g3.reshape(1, -1), be3.reshape(1, -1))
    return out


# speedup vs baseline: 2.9148x; 2.9148x over previous
import functools

import jax
import jax.numpy as jnp
from jax import lax
from jax.experimental import pallas as pl
from jax.experimental.pallas import tpu as pltpu
from jax.experimental.pallas import tpu_sc as plsc

_N = 10000
_E = 320000
_D = 128

_NC = 2
_NS = 16
_NW = _NC * _NS
_STR = 624
_TAIL0 = _STR * _NS
_TAIL = _N - _TAIL0

_sc_mesh = plsc.VectorSubcoreMesh(core_axis_name="c", subcore_axis_name="s")

_CH = 128
_CPT = 80
_EPAD = _NW * _CPT * _CH
_NROW_CHUNKS = _EPAD // _CH
_NREAL = _E // _CH
_ACC_ROWS = _N + 16
_NBUF = 2
_CPH = 40

_DEG_W = 16
_DEG_NBUF = 4


def _deg_body(dst_hbm, ones_hbm, zeros_hbm, deg_out, idx_v, ones_v, acc_sh,
              *sems):
    cid = lax.axis_index("c")
    sid = lax.axis_index("s")
    wid = sid * _NC + cid
    r0 = sid * _STR

    pltpu.sync_copy(dst_hbm.at[pl.ds(wid * _CPT, _CPT)], idx_v)
    pltpu.sync_copy(ones_hbm, ones_v)
    pltpu.sync_copy(zeros_hbm.at[pl.ds(r0, _STR)], acc_sh.at[pl.ds(r0, _STR)])

    @pl.when(sid == 0)
    def _():
        pltpu.sync_copy(zeros_hbm.at[pl.ds(_TAIL0, _TAIL)],
                        acc_sh.at[pl.ds(_TAIL0, _TAIL)])

    plsc.subcore_barrier()

    n_w = jnp.clip(_NREAL - wid * _CPT, 0, _CPT)

    def group(g, carry):
        for b in range(_DEG_NBUF):
            c = g * _DEG_NBUF + b

            @pl.when(g > 0)
            def _():
                pltpu.make_async_copy(
                    ones_v, acc_sh.at[idx_v.at[c - _DEG_NBUF]], sems[b]).wait()

            pltpu.async_copy(ones_v, acc_sh.at[idx_v.at[c]], sems[b], add=True)
        return carry

    lax.fori_loop(0, n_w // _DEG_NBUF, group, 0)

    @pl.when(n_w > 0)
    def _():
        for b in range(_DEG_NBUF):
            c = n_w - _DEG_NBUF + b
            pltpu.make_async_copy(ones_v, acc_sh.at[idx_v.at[c]],
                                  sems[b]).wait()

    plsc.subcore_barrier()
    pltpu.sync_copy(acc_sh.at[pl.ds(r0, _STR)],
                    deg_out.at[cid, pl.ds(r0, _STR)])

    @pl.when(sid == 0)
    def _():
        pltpu.sync_copy(acc_sh.at[pl.ds(_TAIL0, _TAIL)],
                        deg_out.at[cid, pl.ds(_TAIL0, _TAIL)])


_deg_kernel = pl.kernel(
    _deg_body,
    out_type=jax.ShapeDtypeStruct((_NC, _N, _DEG_W), jnp.float32),
    mesh=_sc_mesh,
    scratch_types=[
        pltpu.VMEM((_CPT, _CH), jnp.int32),
        pltpu.VMEM((_CH, _DEG_W), jnp.float32),
        pltpu.VMEM_SHARED((_ACC_ROWS, _DEG_W), jnp.float32),
    ] + [pltpu.SemaphoreType.DMA] * _DEG_NBUF,
)


def _spmm_body(hs_hbm, src_hbm, dst_hbm, zeros_hbm, out_hbm,
               src_v, dst_v, rows_v, acc_sh, *sems):
    cid = lax.axis_index("c")
    sid = lax.axis_index("s")
    wid = sid * _NC + cid
    r0 = sid * _STR
    sg = sems[:_NBUF]
    ss = sems[_NBUF:]

    pltpu.sync_copy(zeros_hbm.at[pl.ds(r0, _STR)], acc_sh.at[pl.ds(r0, _STR)])

    @pl.when(sid == 0)
    def _():
        pltpu.sync_copy(zeros_hbm.at[pl.ds(_TAIL0, _TAIL)],
                        acc_sh.at[pl.ds(_TAIL0, _TAIL)])

    plsc.subcore_barrier()

    n_w = jnp.clip(_NREAL - wid * _CPT, 0, _CPT)

    for phase in range(_CPT // _CPH):
        n_ph = jnp.clip(n_w - phase * _CPH, 0, _CPH)

        @pl.when(n_ph > 0)
        def _():
            base = wid * _CPT + phase * _CPH
            pltpu.sync_copy(src_hbm.at[pl.ds(base, _CPH)], src_v)
            pltpu.sync_copy(dst_hbm.at[pl.ds(base, _CPH)], dst_v)

            def group(g, carry):
                for b in range(_NBUF):
                    c = g * _NBUF + b

                    @pl.when(g > 0)
                    def _():
                        pltpu.make_async_copy(
                            rows_v.at[b], acc_sh.at[dst_v.at[c - _NBUF]],
                            ss[b]).wait()

                    pltpu.async_copy(hs_hbm.at[src_v.at[c]], rows_v.at[b],
                                     sg[b])
                for b in range(_NBUF):
                    c = g * _NBUF + b
                    pltpu.make_async_copy(hs_hbm.at[src_v.at[c]], rows_v.at[b],
                                          sg[b]).wait()
                    pltpu.async_copy(rows_v.at[b], acc_sh.at[dst_v.at[c]],
                                     ss[b], add=True)
                return carry

            lax.fori_loop(0, n_ph // _NBUF, group, 0)
            for b in range(_NBUF):
                c = n_ph - _NBUF + b
                pltpu.make_async_copy(rows_v.at[b], acc_sh.at[dst_v.at[c]],
                                      ss[b]).wait()

    plsc.subcore_barrier()
    pltpu.sync_copy(acc_sh.at[pl.ds(r0, _STR)],
                    out_hbm.at[cid, pl.ds(r0, _STR)])

    @pl.when(sid == 0)
    def _():
        pltpu.sync_copy(acc_sh.at[pl.ds(_TAIL0, _TAIL)],
                        out_hbm.at[cid, pl.ds(_TAIL0, _TAIL)])


_spmm_kernel = pl.kernel(
    _spmm_body,
    out_type=jax.ShapeDtypeStruct((_NC, _N, _D), jnp.float32),
    mesh=_sc_mesh,
    scratch_types=[
        pltpu.VMEM((_CPH, _CH), jnp.int32),
        pltpu.VMEM((_CPH, _CH), jnp.int32),
        pltpu.VMEM((_NBUF, _CH, _D), jnp.float32),
        pltpu.VMEM_SHARED((_ACC_ROWS, _D), jnp.float32),
    ] + [pltpu.SemaphoreType.DMA] * (2 * _NBUF),
)

_RB = 2000
_NB = _N // _RB


def _pre_body(deg_ref, x_ref, dinv_ref, xs_ref):
    deg = deg_ref[0, :, :1] + deg_ref[1, :, :1] + 1.0
    dinv = lax.rsqrt(deg)
    dinv_ref[...] = dinv
    xs_ref[...] = x_ref[...] * dinv


_pre_call = pl.pallas_call(
    _pre_body,
    out_shape=[
        jax.ShapeDtypeStruct((_N, 1), jnp.float32),
        jax.ShapeDtypeStruct((_N, _D), jnp.float32),
    ],
)


def _mm1_body(parts_ref, self_ref, dinv_ref, w_ref, b_ref, t_ref, sums_ref):
    p = (parts_ref[0] + parts_ref[1] + self_ref[...]) * dinv_ref[...]
    t = jnp.dot(p, w_ref[...], preferred_element_type=jnp.float32) + b_ref[...]
    t = jnp.maximum(t, 0.0)
    t_ref[...] = t
    sums_ref[...] = jnp.stack([jnp.sum(t, axis=0), jnp.sum(t * t, axis=0)])[None]


def _mk_mm1(wout):
    return pl.pallas_call(
        _mm1_body,
        grid=(_NB,),
        in_specs=[
            pl.BlockSpec((_NC, _RB, _D), lambda i: (0, i, 0)),
            pl.BlockSpec((_RB, _D), lambda i: (i, 0)),
            pl.BlockSpec((_RB, 1), lambda i: (i, 0)),
            pl.BlockSpec((_D, wout), lambda i: (0, 0)),
            pl.BlockSpec((1, wout), lambda i: (0, 0)),
        ],
        out_specs=[
            pl.BlockSpec((_RB, wout), lambda i: (i, 0)),
            pl.BlockSpec((1, 2, wout), lambda i: (i, 0, 0)),
        ],
        out_shape=[
            jax.ShapeDtypeStruct((_N, wout), jnp.float32),
            jax.ShapeDtypeStruct((_NB, 2, wout), jnp.float32),
        ],
    )


_mm_128 = _mk_mm1(_D)
_mm_256 = _mk_mm1(2 * _D)


def _mm2_body(pa_ref, pb_ref, sa_ref, sb_ref, dinv_ref, w_ref, b_ref,
              t_ref, sums_ref):
    dinv = dinv_ref[...]
    pa = (pa_ref[0] + pa_ref[1] + sa_ref[...]) * dinv
    pb = (pb_ref[0] + pb_ref[1] + sb_ref[...]) * dinv
    p = jnp.concatenate([pa, pb], axis=1)
    t = jnp.dot(p, w_ref[...], preferred_element_type=jnp.float32) + b_ref[...]
    t = jnp.maximum(t, 0.0)
    t_ref[...] = t
    sums_ref[...] = jnp.stack([jnp.sum(t, axis=0), jnp.sum(t * t, axis=0)])[None]


_mm2_256 = pl.pallas_call(
    _mm2_body,
    grid=(_NB,),
    in_specs=[
        pl.BlockSpec((_NC, _RB, _D), lambda i: (0, i, 0)),
        pl.BlockSpec((_NC, _RB, _D), lambda i: (0, i, 0)),
        pl.BlockSpec((_RB, _D), lambda i: (i, 0)),
        pl.BlockSpec((_RB, _D), lambda i: (i, 0)),
        pl.BlockSpec((_RB, 1), lambda i: (i, 0)),
        pl.BlockSpec((2 * _D, 2 * _D), lambda i: (0, 0)),
        pl.BlockSpec((1, 2 * _D), lambda i: (0, 0)),
    ],
    out_specs=[
        pl.BlockSpec((_RB, 2 * _D), lambda i: (i, 0)),
        pl.BlockSpec((1, 2, 2 * _D), lambda i: (i, 0, 0)),
    ],
    out_shape=[
        jax.ShapeDtypeStruct((_N, 2 * _D), jnp.float32),
        jax.ShapeDtypeStruct((_NB, 2, 2 * _D), jnp.float32),
    ],
)


def _bn_core(t_ref, sums_ref, g_ref, be_ref):
    s = jnp.sum(sums_ref[...], axis=0)
    m = s[0] * (1.0 / _N)
    v = s[1] * (1.0 / _N) - m * m
    scale = g_ref[...] * lax.rsqrt(v + 1e-5)[None, :]
    return (t_ref[...] - m[None, :]) * scale + be_ref[...]


def _bn_scale_body(t_ref, sums_ref, g_ref, be_ref, dinv_ref, o_ref):
    o_ref[...] = _bn_core(t_ref, sums_ref, g_ref, be_ref) * dinv_ref[...]


def _bn_split_body(t_ref, sums_ref, g_ref, be_ref, dinv_ref, oa_ref, ob_ref):
    h = _bn_core(t_ref, sums_ref, g_ref, be_ref) * dinv_ref[...]
    oa_ref[...] = h[:, :_D]
    ob_ref[...] = h[:, _D:]


def _bn_final_body(t_ref, sums_ref, g_ref, be_ref, o_ref):
    o_ref[...] = _bn_core(t_ref, sums_ref, g_ref, be_ref)


def _bn_in_specs(wout, with_dinv):
    specs = [
        pl.BlockSpec((_RB, wout), lambda i: (i, 0)),
        pl.BlockSpec((_NB, 2, wout), lambda i: (0, 0, 0)),
        pl.BlockSpec((1, wout), lambda i: (0, 0)),
        pl.BlockSpec((1, wout), lambda i: (0, 0)),
    ]
    if with_dinv:
        specs.append(pl.BlockSpec((_RB, 1), lambda i: (i, 0)))
    return specs


_bn_scale_128 = pl.pallas_call(
    _bn_scale_body,
    grid=(_NB,),
    in_specs=_bn_in_specs(_D, True),
    out_specs=pl.BlockSpec((_RB, _D), lambda i: (i, 0)),
    out_shape=jax.ShapeDtypeStruct((_N, _D), jnp.float32),
)

_bn_split_256 = pl.pallas_call(
    _bn_split_body,
    grid=(_NB,),
    in_specs=_bn_in_specs(2 * _D, True),
    out_specs=[
        pl.BlockSpec((_RB, _D), lambda i: (i, 0)),
        pl.BlockSpec((_RB, _D), lambda i: (i, 0)),
    ],
    out_shape=[
        jax.ShapeDtypeStruct((_N, _D), jnp.float32),
        jax.ShapeDtypeStruct((_N, _D), jnp.float32),
    ],
)

_bn_final_256 = pl.pallas_call(
    _bn_final_body,
    grid=(_NB,),
    in_specs=_bn_in_specs(2 * _D, False),
    out_specs=pl.BlockSpec((_RB, 2 * _D), lambda i: (i, 0)),
    out_shape=jax.ShapeDtypeStruct((_N, 2 * _D), jnp.float32),
)


def kernel(x, edge_index, W1, b1, g1, be1, W2, b2, g2, be2, W3, b3, g3, be3):
    pad = _EPAD - _E
    src = jnp.concatenate([edge_index[0],
                           jnp.zeros((pad,), jnp.int32)]).reshape(
                               _NROW_CHUNKS, _CH)
    dst = jnp.concatenate([edge_index[1],
                           jnp.full((pad,), _N, jnp.int32)]).reshape(
                               _NROW_CHUNKS, _CH)

    zeros = jnp.zeros((_N, _D), jnp.float32)
    ones16 = jnp.ones((_CH, _DEG_W), jnp.float32)
    deg_parts = _deg_kernel(dst, ones16, zeros[:, :_DEG_W])
    dinv, xs = _pre_call(deg_parts, x)

    s0 = _spmm_kernel(xs, src, dst, zeros)
    t1, sums1 = _mm_128(s0, xs, dinv, W1, b1.reshape(1, -1))
    hs1 = _bn_scale_128(t1, sums1, g1.reshape(1, -1), be1.reshape(1, -1), dinv)

    s1 = _spmm_kernel(hs1, src, dst, zeros)
    t2, sums2 = _mm_256(s1, hs1, dinv, W2, b2.reshape(1, -1))
    hs2a, hs2b = _bn_split_256(t2, sums2, g2.reshape(1, -1),
                               be2.reshape(1, -1), dinv)

    s2a = _spmm_kernel(hs2a, src, dst, zeros)
    s2b = _spmm_kernel(hs2b, src, dst, zeros)
    t3, sums3 = _mm2_256(s2a, s2b, hs2a, hs2b, dinv, W3, b3.reshape(1, -1))
    out = _bn_final_256(t3, sums3, g3.reshape(1, -1), be3.reshape(1, -1))
    return out


# speedup vs baseline: 2.9536x; 1.0133x over previous
import functools

import jax
import jax.numpy as jnp
from jax import lax
from jax.experimental import pallas as pl
from jax.experimental.pallas import tpu as pltpu
from jax.experimental.pallas import tpu_sc as plsc

_N = 10000
_E = 320000
_D = 128

_NC = 2
_NS = 16
_NW = _NC * _NS
_STR = 624
_TAIL0 = _STR * _NS
_TAIL = _N - _TAIL0

_sc_mesh = plsc.VectorSubcoreMesh(core_axis_name="c", subcore_axis_name="s")

_CH = 128
_CPT = 80
_EPAD = _NW * _CPT * _CH
_NROW_CHUNKS = _EPAD // _CH
_NREAL = _E // _CH
_ACC_ROWS = _N + 16
_NBUF = 2
_CPH = 40

_DEG_W = 16
_DEG_NBUF = 4


def _deg_body(dst_hbm, ones_hbm, zeros_hbm, deg_out, idx_v, ones_v, acc_sh,
              *sems):
    cid = lax.axis_index("c")
    sid = lax.axis_index("s")
    wid = sid * _NC + cid
    r0 = sid * _STR

    pltpu.sync_copy(dst_hbm.at[pl.ds(wid * _CPT, _CPT)], idx_v)
    pltpu.sync_copy(ones_hbm, ones_v)
    pltpu.sync_copy(zeros_hbm.at[pl.ds(r0, _STR)], acc_sh.at[pl.ds(r0, _STR)])

    @pl.when(sid == 0)
    def _():
        pltpu.sync_copy(zeros_hbm.at[pl.ds(_TAIL0, _TAIL)],
                        acc_sh.at[pl.ds(_TAIL0, _TAIL)])

    plsc.subcore_barrier()

    n_w = jnp.clip(_NREAL - wid * _CPT, 0, _CPT)

    def group(g, carry):
        for b in range(_DEG_NBUF):
            c = g * _DEG_NBUF + b

            @pl.when(g > 0)
            def _():
                pltpu.make_async_copy(
                    ones_v, acc_sh.at[idx_v.at[c - _DEG_NBUF]], sems[b]).wait()

            pltpu.async_copy(ones_v, acc_sh.at[idx_v.at[c]], sems[b], add=True)
        return carry

    lax.fori_loop(0, n_w // _DEG_NBUF, group, 0)

    @pl.when(n_w > 0)
    def _():
        for b in range(_DEG_NBUF):
            c = n_w - _DEG_NBUF + b
            pltpu.make_async_copy(ones_v, acc_sh.at[idx_v.at[c]],
                                  sems[b]).wait()

    plsc.subcore_barrier()
    pltpu.sync_copy(acc_sh.at[pl.ds(r0, _STR)],
                    deg_out.at[cid, pl.ds(r0, _STR)])

    @pl.when(sid == 0)
    def _():
        pltpu.sync_copy(acc_sh.at[pl.ds(_TAIL0, _TAIL)],
                        deg_out.at[cid, pl.ds(_TAIL0, _TAIL)])


_deg_kernel = pl.kernel(
    _deg_body,
    out_type=jax.ShapeDtypeStruct((_NC, _N, _DEG_W), jnp.float32),
    mesh=_sc_mesh,
    scratch_types=[
        pltpu.VMEM((_CPT, _CH), jnp.int32),
        pltpu.VMEM((_CH, _DEG_W), jnp.float32),
        pltpu.VMEM_SHARED((_ACC_ROWS, _DEG_W), jnp.float32),
    ] + [pltpu.SemaphoreType.DMA] * _DEG_NBUF,
)


def _spmm_body(hs_hbm, src_hbm, dst_hbm, zeros_hbm, out_hbm,
               src_v, dst_v, rows_v, acc_sh, *sems):
    cid = lax.axis_index("c")
    sid = lax.axis_index("s")
    wid = sid * _NC + cid
    r0 = sid * _STR
    sg = sems[:_NBUF]
    ss = sems[_NBUF:]

    pltpu.sync_copy(zeros_hbm.at[pl.ds(r0, _STR)], acc_sh.at[pl.ds(r0, _STR)])

    @pl.when(sid == 0)
    def _():
        pltpu.sync_copy(zeros_hbm.at[pl.ds(_TAIL0, _TAIL)],
                        acc_sh.at[pl.ds(_TAIL0, _TAIL)])

    plsc.subcore_barrier()

    n_w = jnp.clip(_NREAL - wid * _CPT, 0, _CPT)

    for phase in range(_CPT // _CPH):
        n_ph = jnp.clip(n_w - phase * _CPH, 0, _CPH)

        @pl.when(n_ph > 0)
        def _():
            base = wid * _CPT + phase * _CPH
            pltpu.sync_copy(src_hbm.at[pl.ds(base, _CPH)], src_v)
            pltpu.sync_copy(dst_hbm.at[pl.ds(base, _CPH)], dst_v)

            def group(g, carry):
                for b in range(_NBUF):
                    c = g * _NBUF + b

                    @pl.when(g > 0)
                    def _():
                        pltpu.make_async_copy(
                            rows_v.at[b], acc_sh.at[dst_v.at[c - _NBUF]],
                            ss[b]).wait()

                    pltpu.async_copy(hs_hbm.at[src_v.at[c]], rows_v.at[b],
                                     sg[b])
                for b in range(_NBUF):
                    c = g * _NBUF + b
                    pltpu.make_async_copy(hs_hbm.at[src_v.at[c]], rows_v.at[b],
                                          sg[b]).wait()
                    pltpu.async_copy(rows_v.at[b], acc_sh.at[dst_v.at[c]],
                                     ss[b], add=True)
                return carry

            lax.fori_loop(0, n_ph // _NBUF, group, 0)
            for b in range(_NBUF):
                c = n_ph - _NBUF + b
                pltpu.make_async_copy(rows_v.at[b], acc_sh.at[dst_v.at[c]],
                                      ss[b]).wait()

    plsc.subcore_barrier()
    pltpu.sync_copy(acc_sh.at[pl.ds(r0, _STR)],
                    out_hbm.at[cid, pl.ds(r0, _STR)])

    @pl.when(sid == 0)
    def _():
        pltpu.sync_copy(acc_sh.at[pl.ds(_TAIL0, _TAIL)],
                        out_hbm.at[cid, pl.ds(_TAIL0, _TAIL)])


_spmm_kernel = pl.kernel(
    _spmm_body,
    out_type=jax.ShapeDtypeStruct((_NC, _N, _D), jnp.float32),
    mesh=_sc_mesh,
    scratch_types=[
        pltpu.VMEM((_CPH, _CH), jnp.int32),
        pltpu.VMEM((_CPH, _CH), jnp.int32),
        pltpu.VMEM((_NBUF, _CH, _D), jnp.float32),
        pltpu.VMEM_SHARED((_ACC_ROWS, _D), jnp.float32),
    ] + [pltpu.SemaphoreType.DMA] * (2 * _NBUF),
)

_RB = 2000
_NB = _N // _RB


def _pre_body(deg_ref, x_ref, dinv_ref, xs_ref):
    deg = deg_ref[0, :, :1] + deg_ref[1, :, :1] + 1.0
    dinv = lax.rsqrt(deg)
    dinv_ref[...] = dinv
    xs_ref[...] = x_ref[...] * dinv


_pre_call = pl.pallas_call(
    _pre_body,
    out_shape=[
        jax.ShapeDtypeStruct((_N, 1), jnp.float32),
        jax.ShapeDtypeStruct((_N, _D), jnp.float32),
    ],
)


def _mk_layer(wout, two_groups, split, final):
    def body(*refs):
        n_in = (7 if two_groups else 5) if final else (8 if two_groups else 6)
        it = iter(refs)
        pa_ref = next(it)
        pb_ref = next(it) if two_groups else None
        sa_ref = next(it)
        sb_ref = next(it) if two_groups else None
        dinv_ref = next(it)
        w_ref = next(it)
        b_ref = next(it)
        g_ref = next(it)
        be_ref = next(it)
        if split:
            oa_ref = next(it)
            ob_ref = next(it)
        else:
            o_ref = next(it)
        t_s = next(it)
        sums_s = next(it)

        p = pl.program_id(0)
        i = pl.program_id(1)

        @pl.when(p == 0)
        def _():
            dinv = dinv_ref[...]
            pblk = (pa_ref[0] + pa_ref[1] + sa_ref[...]) * dinv
            if two_groups:
                pb = (pb_ref[0] + pb_ref[1] + sb_ref[...]) * dinv
                pblk = jnp.concatenate([pblk, pb], axis=1)
            t = jnp.dot(pblk, w_ref[...],
                        preferred_element_type=jnp.float32) + b_ref[...]
            t = jnp.maximum(t, 0.0)
            t_s[pl.ds(i * _RB, _RB), :] = t
            blk = jnp.stack([jnp.sum(t, axis=0), jnp.sum(t * t, axis=0)])

            @pl.when(i == 0)
            def _():
                sums_s[...] = blk

            @pl.when(i > 0)
            def _():
                sums_s[...] += blk

        @pl.when(p == 1)
        def _():
            s = sums_s[...]
            m = s[0] * (1.0 / _N)
            v = s[1] * (1.0 / _N) - m * m
            scale = g_ref[...] * lax.rsqrt(v + 1e-5)[None, :]
            h = ((t_s[pl.ds(i * _RB, _RB), :] - m[None, :]) * scale
                 + be_ref[...])
            if not final:
                h = h * dinv_ref[...]
            if split:
                oa_ref[...] = h[:, :_D]
                ob_ref[...] = h[:, _D:]
            else:
                o_ref[...] = h

    parts_spec = pl.BlockSpec((_NC, _RB, _D),
                              lambda p, i: (0, i * (1 - p), 0))
    row_spec = pl.BlockSpec((_RB, _D), lambda p, i: (i * (1 - p), 0))
    in_specs = [parts_spec] + ([parts_spec] if two_groups else []) \
        + [row_spec] + ([row_spec] if two_groups else []) \
        + [pl.BlockSpec((_RB, 1), lambda p, i: (i, 0)),
           pl.BlockSpec((2 * _D if two_groups else _D, wout),
                        lambda p, i: (0, 0)),
           pl.BlockSpec((1, wout), lambda p, i: (0, 0)),
           pl.BlockSpec((1, wout), lambda p, i: (0, 0)),
           pl.BlockSpec((1, wout), lambda p, i: (0, 0))]
    if split:
        out_specs = [pl.BlockSpec((_RB, _D), lambda p, i: (i * p, 0)),
                     pl.BlockSpec((_RB, _D), lambda p, i: (i * p, 0))]
        out_shape = [jax.ShapeDtypeStruct((_N, _D), jnp.float32),
                     jax.ShapeDtypeStruct((_N, _D), jnp.float32)]
    else:
        out_specs = pl.BlockSpec((_RB, wout), lambda p, i: (i * p, 0))
        out_shape = jax.ShapeDtypeStruct((_N, wout), jnp.float32)
    return pl.pallas_call(
        body,
        grid=(2, _NB),
        in_specs=in_specs,
        out_specs=out_specs,
        out_shape=out_shape,
        scratch_shapes=[pltpu.VMEM((_N, wout), jnp.float32),
                        pltpu.VMEM((2, wout), jnp.float32)],
    )


_layer1 = _mk_layer(_D, False, False, False)
_layer2 = _mk_layer(2 * _D, False, True, False)
_layer3 = _mk_layer(2 * _D, True, False, True)


def kernel(x, edge_index, W1, b1, g1, be1, W2, b2, g2, be2, W3, b3, g3, be3):
    pad = _EPAD - _E
    src = jnp.concatenate([edge_index[0],
                           jnp.zeros((pad,), jnp.int32)]).reshape(
                               _NROW_CHUNKS, _CH)
    dst = jnp.concatenate([edge_index[1],
                           jnp.full((pad,), _N, jnp.int32)]).reshape(
                               _NROW_CHUNKS, _CH)

    zeros = jnp.zeros((_N, _D), jnp.float32)
    ones16 = jnp.ones((_CH, _DEG_W), jnp.float32)
    deg_parts = _deg_kernel(dst, ones16, zeros[:, :_DEG_W])
    dinv, xs = _pre_call(deg_parts, x)

    s0 = _spmm_kernel(xs, src, dst, zeros)
    hs1 = _layer1(s0, xs, dinv, W1, b1.reshape(1, -1), g1.reshape(1, -1),
                  be1.reshape(1, -1))

    s1 = _spmm_kernel(hs1, src, dst, zeros)
    hs2a, hs2b = _layer2(s1, hs1, dinv, W2, b2.reshape(1, -1),
                         g2.reshape(1, -1), be2.reshape(1, -1))

    s2a = _spmm_kernel(hs2a, src, dst, zeros)
    s2b = _spmm_kernel(hs2b, src, dst, zeros)
    out = _layer3(s2a, s2b, hs2a, hs2b, dinv, W3, b3.reshape(1, -1),
                  g3.reshape(1, -1), be3.reshape(1, -1))
    return out
